# Initial kernel scaffold; baseline (speedup 1.0000x reference)
import jax
import jax.numpy as jnp
from jax.experimental import pallas as pl


def kernel(input, weight):
    raise NotImplementedError("write your pallas kernel here")


# speedup vs baseline: 1.0230x; 1.0230x over previous
import functools

import jax
import jax.numpy as jnp
from jax import lax
from jax.experimental import pallas as pl
from jax.experimental.pallas import tpu as pltpu
from jax.experimental.pallas import tpu_sc as plsc

EMBED_DIM = 32
NUM_CORES = 2
NUM_SUBCORES = 16
NUM_WORKERS = NUM_CORES * NUM_SUBCORES
CHUNK = 128


@functools.lru_cache(maxsize=None)
def _make_gather(B: int, V: int, D: int):
    assert B % (NUM_WORKERS * CHUNK) == 0
    b_per_w = B // NUM_WORKERS
    n_chunks = b_per_w // CHUNK
    mesh = plsc.VectorSubcoreMesh(core_axis_name="c", subcore_axis_name="s")

    @functools.partial(
        pl.kernel,
        mesh=mesh,
        out_type=jax.ShapeDtypeStruct((B, D), jnp.float32),
        scratch_types=[
            pltpu.VMEM((n_chunks, CHUNK), jnp.int32),
            pltpu.VMEM((CHUNK, D), jnp.float32),
            pltpu.SemaphoreType.DMA,
        ],
        compiler_params=pltpu.CompilerParams(use_tc_tiling_on_sc=False),
    )
    def gather_kernel(idx_hbm, table_hbm, out_hbm, idx_v, rows_v, sem):
        wid = lax.axis_index("s") * NUM_CORES + lax.axis_index("c")
        pltpu.sync_copy(idx_hbm.at[wid], idx_v)

        def body(j, carry):
            pltpu.async_copy(table_hbm.at[idx_v.at[j]], rows_v, sem).wait()
            base = wid * b_per_w + j * CHUNK
            pltpu.sync_copy(rows_v, out_hbm.at[pl.ds(base, CHUNK)])
            return carry

        lax.fori_loop(0, n_chunks, body, 0)

    return gather_kernel


def kernel(input, weight):
    B = input.shape[0] * input.shape[1]
    V, D = weight.shape
    idx = input.reshape(NUM_WORKERS, B // (NUM_WORKERS * CHUNK), CHUNK)
    idx = idx.astype(jnp.int32)
    out = _make_gather(B, V, D)(idx, weight)
    return out.reshape(input.shape[0], input.shape[1], D)


# speedup vs baseline: 1.1106x; 1.0856x over previous
import functools

import jax
import jax.numpy as jnp
from jax import lax
from jax.experimental import pallas as pl
from jax.experimental.pallas import tpu as pltpu
from jax.experimental.pallas import tpu_sc as plsc

EMBED_DIM = 32
NUM_CORES = 2
NUM_SUBCORES = 16
NUM_WORKERS = NUM_CORES * NUM_SUBCORES
CHUNK = 128
K = 8


@functools.lru_cache(maxsize=None)
def _make_gather(B: int, V: int, D: int):
    assert B % (NUM_WORKERS * CHUNK) == 0
    b_per_w = B // NUM_WORKERS
    n_chunks = b_per_w // CHUNK
    assert n_chunks % K == 0
    n_groups = n_chunks // K
    n_stages = n_groups + 1
    mesh = plsc.VectorSubcoreMesh(core_axis_name="c", subcore_axis_name="s")

    @functools.partial(
        pl.kernel,
        mesh=mesh,
        out_type=jax.ShapeDtypeStruct((B, D), jnp.float32),
        scratch_types=[
            pltpu.VMEM((n_chunks, CHUNK), jnp.int32),
            pltpu.VMEM((K, CHUNK, D), jnp.float32),
            pltpu.VMEM((K, CHUNK, D), jnp.float32),
            pltpu.SemaphoreType.DMA,
            pltpu.SemaphoreType.DMA,
            pltpu.SemaphoreType.DMA,
            pltpu.SemaphoreType.DMA,
        ],
        compiler_params=pltpu.CompilerParams(use_tc_tiling_on_sc=False),
    )
    def gather_kernel(idx_hbm, table_hbm, out_hbm, idx_v, rows0, rows1,
                      gsem0, gsem1, ssem0, ssem1):
        wid = lax.axis_index("s") * NUM_CORES + lax.axis_index("c")
        out_base = wid * b_per_w
        pltpu.sync_copy(idx_hbm.at[wid], idx_v)

        banks = ((rows0, gsem0, ssem0), (rows1, gsem1, ssem1))

        def fire_gathers(bank, g):
            rows, gsem, _ = banks[bank]
            for k in range(K):
                c = g * K + k
                pltpu.async_copy(table_hbm.at[idx_v.at[c]], rows.at[k], gsem)

        def drain_gathers(bank, g):
            rows, gsem, _ = banks[bank]
            for k in range(K):
                c = g * K + k
                pltpu.make_async_copy(table_hbm.at[idx_v.at[c]],
                                      rows.at[k], gsem).wait()

        def fire_stores(bank, g):
            rows, _, ssem = banks[bank]
            for k in range(K):
                c = g * K + k
                pltpu.async_copy(rows.at[k],
                                 out_hbm.at[pl.ds(out_base + c * CHUNK, CHUNK)],
                                 ssem)

        def drain_stores(bank, g):
            rows, _, ssem = banks[bank]
            for k in range(K):
                c = g * K + k
                pltpu.make_async_copy(rows.at[k],
                                      out_hbm.at[pl.ds(out_base + c * CHUNK,
                                                       CHUNK)],
                                      ssem).wait()

        fire_gathers(0, 0)

        def stage(t, bank):
            other = 1 - bank
            @pl.when(t < n_groups)
            def _():
                drain_gathers(bank, t)
                fire_stores(bank, t)

            @pl.when(t >= 1)
            def _():
                drain_stores(other, t - 1)

            @pl.when(t + 1 < n_groups)
            def _():
                fire_gathers(other, t + 1)

        def body(i, carry):
            stage(2 * i, 0)
            stage(2 * i + 1, 1)
            return carry

        lax.fori_loop(0, (n_stages + 1) // 2, body, 0)

    return gather_kernel


def kernel(input, weight):
    B = input.shape[0] * input.shape[1]
    V, D = weight.shape
    idx = input.reshape(NUM_WORKERS, B // (NUM_WORKERS * CHUNK), CHUNK)
    idx = idx.astype(jnp.int32)
    out = _make_gather(B, V, D)(idx, weight)
    return out.reshape(input.shape[0], input.shape[1], D)
